# baseline (speedup 1.0000x reference)
import functools

import jax
import jax.numpy as jnp
from jax import lax
from jax.experimental import pallas as pl
from jax.experimental.pallas import tpu as pltpu
from jax.experimental.pallas import tpu_sc as plsc

N_IN = 10000
N_OUT = 10000
E = 320000
D = 128
K = 4
K3 = K * K * K
DEG = 32

NC = 2
NS = 16
NW = NC * NS
VPT = 320
VOXPAD = NW * VPT
EPT = VPT * DEG
EPAD = VOXPAD * DEG

SROW = K3 * D

CORNERS = [(dx, dy, dz) for dx in (0, 1) for dy in (0, 1) for dz in (0, 1)]

_f32 = jnp.float32
_i32 = jnp.int32


def _sqrt_newton(r2):
    bi = lax.bitcast_convert_type(r2, _i32)
    bi = (bi >> 1) + 0x1FBD1DF5
    y = lax.bitcast_convert_type(bi, _f32)
    for _ in range(3):
        y = 0.5 * (y + r2 / y)
    return y


def _sc_body(vpt_a, vpt_b,
             feats, idxp, scalep, distp, px, py, pz, ox, oy, oz,
             s_out, norm_out,
             idx_v, sc_v, di_v, px_v, py_v, pz_v, ox_v, oy_v, oz_v,
             fbufs, accs, wbuf, cbuf, nbuf,
             gsems, fsems):
    cid = lax.axis_index("c")
    sid = lax.axis_index("s")

    pltpu.sync_copy(px, px_v)
    pltpu.sync_copy(py, py_v)
    pltpu.sync_copy(pz, pz_v)

    def fire_gather(vl, fbuf, gsem):
        pltpu.async_copy(feats.at[idx_v.at[pl.ds(vl * DEG, DEG)]], fbuf, gsem)

    def wait_gather(fbuf, gsem):
        pltpu.make_async_copy(feats.at[pl.ds(0, DEG)], fbuf, gsem).wait()

    def wait_flush(acc, fsem):
        pltpu.make_async_copy(acc, s_out.at[0], fsem).wait()

    def zero_acc(acc):
        z = jnp.zeros((16,), _f32)

        def zbody(j, _):
            for i in range(16):
                acc[pl.ds(j * 256 + i * 16, 16)] = z
            return 0

        lax.fori_loop(0, SROW // 256, zbody, 0)

    def geom(vl, g):
        eb = vl * DEG + g * 16
        ii = idx_v[pl.ds(eb, 16)]
        rx = plsc.load_gather(px_v, [ii]) - ox_v[pl.ds(vl, 16)][0]
        ry = plsc.load_gather(py_v, [ii]) - oy_v[pl.ds(vl, 16)][0]
        rz = plsc.load_gather(pz_v, [ii]) - oz_v[pl.ds(vl, 16)][0]
        r2 = rx * rx + ry * ry + rz * rz + 1e-12
        rnorm = _sqrt_newton(r2)
        linf = jnp.maximum(jnp.maximum(jnp.abs(rx), jnp.abs(ry)), jnp.abs(rz)) + 1e-12
        cc = rnorm / linf

        def axis_coords(r):
            q = jnp.minimum(jnp.maximum(r * cc, -1.0), 1.0)
            t = (q + 1.0) * (0.5 * (K - 1))
            t0 = jnp.minimum(t.astype(_i32), K - 2)
            f = jnp.minimum(jnp.maximum(t - t0.astype(_f32), 0.0), 1.0)
            return t0, f

        t0x, fx = axis_coords(rx)
        t0y, fy = axis_coords(ry)
        t0z, fz = axis_coords(rz)

        dd = di_v[pl.ds(eb, 16)]
        x6 = jnp.minimum(jnp.maximum(1.0 - dd * dd, 0.0), 1.0)
        imp = sc_v[pl.ds(eb, 16)] * (x6 * x6 * x6)

        wx = ((1.0 - fx) * imp, fx * imp)
        wy = (1.0 - fy, fy)
        wz = (1.0 - fz, fz)
        base = (t0x * (K * K) + t0y * K + t0z) * D
        lanes8 = lax.broadcasted_iota(_i32, (16,), 0) * 8 + (g * 128)
        for k, (dx, dy, dz) in enumerate(CORNERS):
            plsc.store_scatter(wbuf, [lanes8 + k], wx[dx] * wy[dy] * wz[dz])
            plsc.store_scatter(cbuf, [lanes8 + k],
                               base + (dx * (K * K * D) + dy * (K * D) + dz * D))
        return jnp.sum(imp)

    def accumulate(fbuf, acc):
        def pbody(p, _):
            wv = wbuf[pl.ds(p * 16, 16)]
            cv = cbuf[pl.ds(p * 16, 16)]
            for j in range(2):
                e = p * 2 + j
                fv = [fbuf[e, pl.ds(i * 16, 16)] for i in range(8)]
                for k in range(8):
                    w = wv[j * 8 + k]
                    co = cv[j * 8 + k]
                    for i in range(8):
                        plsc.addupdate(acc.at[pl.ds(co + i * 16, 16)], w * fv[i])
            return 0

        lax.fori_loop(0, DEG // 2, pbody, 0)

    NB = 4

    def run(vpt, v0):
        ept = vpt * DEG
        e0 = v0 * DEG
        pltpu.sync_copy(idxp.at[pl.ds(e0, ept)], idx_v.at[pl.ds(0, ept)])
        pltpu.sync_copy(scalep.at[pl.ds(e0, ept)], sc_v.at[pl.ds(0, ept)])
        pltpu.sync_copy(distp.at[pl.ds(e0, ept)], di_v.at[pl.ds(0, ept)])
        pltpu.sync_copy(ox.at[pl.ds(v0, vpt)], ox_v.at[pl.ds(0, vpt)])
        pltpu.sync_copy(oy.at[pl.ds(v0, vpt)], oy_v.at[pl.ds(0, vpt)])
        pltpu.sync_copy(oz.at[pl.ds(v0, vpt)], oz_v.at[pl.ds(0, vpt)])

        def process(it, j):
            vl = it * NB + j
            fbuf, acc = fbufs[j], accs[j]
            gsem, fsem = gsems[j], fsems[j]

            @pl.when(it > 0)
            def _():
                wait_flush(acc, fsem)

            zero_acc(acc)
            wait_gather(fbuf, gsem)
            nsum = geom(vl, 0) + geom(vl, 1)
            lane0 = lax.broadcasted_iota(_i32, (16,), 0) == 0
            plsc.store_scatter(nbuf, [jnp.zeros((16,), _i32) + vl],
                               jnp.zeros((16,), _f32) + nsum, mask=lane0)
            accumulate(fbuf, acc)
            pltpu.async_copy(acc, s_out.at[v0 + vl], fsem)

            @pl.when(vl + NB < vpt)
            def _():
                fire_gather(vl + NB, fbuf, gsem)

        for j in range(NB):
            fire_gather(j, fbufs[j], gsems[j])

        def vbody(it, _):
            for j in range(NB):
                process(it, j)
            return 0

        lax.fori_loop(0, vpt // NB, vbody, 0)

        for j in range(NB):
            wait_flush(accs[j], fsems[j])
        pltpu.sync_copy(nbuf.at[pl.ds(0, vpt)], norm_out.at[pl.ds(v0, vpt)])

    if vpt_a == vpt_b:
        run(vpt_a, (sid * NC + cid) * vpt_a)
    else:
        @pl.when(cid == 0)
        def _():
            run(vpt_a, sid * vpt_a)

        @pl.when(cid == 1)
        def _():
            run(vpt_b, NS * vpt_a + sid * vpt_b)


def _tc_body(s_ref, w_ref, n_ref, b_ref, o_ref):
    acc = jnp.dot(s_ref[...], w_ref[...], preferred_element_type=_f32)
    n = jnp.maximum(n_ref[...], 1e-8)
    o_ref[...] = jnp.maximum(acc / n + b_ref[...], 0.0)


def kernel(feats, inp_points, out_points, out_extents, scale_compat,
           neighbors_index, neighbors_distance, neighbors_row_splits, W, b):
    inv_r = 2.0 / out_extents[0]

    def col_pad(a, j, n):
        c = a[:, j] * inv_r
        return jnp.pad(c, (0, n - c.shape[0]))

    px = col_pad(inp_points, 0, VOXPAD)
    py = col_pad(inp_points, 1, VOXPAD)
    pz = col_pad(inp_points, 2, VOXPAD)
    ox = col_pad(out_points, 0, VOXPAD)
    oy = col_pad(out_points, 1, VOXPAD)
    oz = col_pad(out_points, 2, VOXPAD)

    idxp = jnp.pad(neighbors_index, (0, EPAD - E))
    scalep = jnp.pad(scale_compat, (0, EPAD - E))
    distp = jnp.pad(neighbors_distance, (0, EPAD - E))

    VPT_A = 360
    VPT_B = 640 - VPT_A
    VPT_MAX = max(VPT_A, VPT_B)
    sc_call = pl.kernel(
        functools.partial(_sc_body, VPT_A, VPT_B),
        mesh=plsc.VectorSubcoreMesh(core_axis_name="c", subcore_axis_name="s"),
        compiler_params=pltpu.CompilerParams(needs_layout_passes=False),
        out_type=[
            jax.ShapeDtypeStruct((VOXPAD, SROW), _f32),
            jax.ShapeDtypeStruct((VOXPAD,), _f32),
        ],
        scratch_types=[
            pltpu.VMEM((VPT_MAX * DEG,), _i32),
            pltpu.VMEM((VPT_MAX * DEG,), _f32),
            pltpu.VMEM((VPT_MAX * DEG,), _f32),
            pltpu.VMEM((VOXPAD,), _f32),
            pltpu.VMEM((VOXPAD,), _f32),
            pltpu.VMEM((VOXPAD,), _f32),
            pltpu.VMEM((VPT_MAX + 16,), _f32),
            pltpu.VMEM((VPT_MAX + 16,), _f32),
            pltpu.VMEM((VPT_MAX + 16,), _f32),
            [pltpu.VMEM((DEG, D), _f32) for _ in range(4)],
            [pltpu.VMEM((SROW,), _f32) for _ in range(4)],
            pltpu.VMEM((256,), _f32),
            pltpu.VMEM((256,), _i32),
            pltpu.VMEM((VPT_MAX,), _f32),
            [pltpu.SemaphoreType.DMA for _ in range(4)],
            [pltpu.SemaphoreType.DMA for _ in range(4)],
        ],
    )
    s, norm = sc_call(feats, idxp, scalep, distp, px, py, pz, ox, oy, oz)

    MBLK = 512
    out = pl.pallas_call(
        _tc_body,
        grid=(VOXPAD // MBLK,),
        in_specs=[
            pl.BlockSpec((MBLK, SROW), lambda i: (i, 0)),
            pl.BlockSpec((SROW, D), lambda i: (0, 0)),
            pl.BlockSpec((MBLK, 1), lambda i: (i, 0)),
            pl.BlockSpec((1, D), lambda i: (0, 0)),
        ],
        out_specs=pl.BlockSpec((MBLK, D), lambda i: (i, 0)),
        out_shape=jax.ShapeDtypeStruct((VOXPAD, D), _f32),
    )(s, W.reshape(SROW, D), norm.reshape(VOXPAD, 1), b.reshape(1, D))

    return out[:N_OUT]


# speedup vs baseline: 1.0052x; 1.0052x over previous
import functools

import jax
import jax.numpy as jnp
from jax import lax
from jax.experimental import pallas as pl
from jax.experimental.pallas import tpu as pltpu
from jax.experimental.pallas import tpu_sc as plsc

N_IN = 10000
N_OUT = 10000
E = 320000
D = 128
K = 4
K3 = K * K * K
DEG = 32

NC = 2
NS = 16
NW = NC * NS
VPT = 320
VOXPAD = NW * VPT
EPT = VPT * DEG
EPAD = VOXPAD * DEG

SROW = K3 * D

CORNERS = [(dx, dy, dz) for dx in (0, 1) for dy in (0, 1) for dz in (0, 1)]

_f32 = jnp.float32
_i32 = jnp.int32


def _sqrt_newton(r2):
    bi = lax.bitcast_convert_type(r2, _i32)
    bi = (bi >> 1) + 0x1FBD1DF5
    y = lax.bitcast_convert_type(bi, _f32)
    for _ in range(3):
        y = 0.5 * (y + r2 / y)
    return y


def _sc_body(vpt_a, vpt_b,
             feats, idxp, scalep, distp, px, py, pz, ox, oy, oz,
             s_out, norm_out,
             idx_v, sc_v, di_v, px_v, py_v, pz_v, ox_v, oy_v, oz_v,
             fbufs, accs, wbuf, cbuf, nbuf,
             gsems, fsems):
    cid = lax.axis_index("c")
    sid = lax.axis_index("s")

    pltpu.sync_copy(px, px_v)
    pltpu.sync_copy(py, py_v)
    pltpu.sync_copy(pz, pz_v)

    def fire_gather(vl, fbuf, gsem):
        pltpu.async_copy(feats.at[idx_v.at[pl.ds(vl * DEG, DEG)]], fbuf, gsem)

    def wait_gather(fbuf, gsem):
        pltpu.make_async_copy(feats.at[pl.ds(0, DEG)], fbuf, gsem).wait()

    def wait_flush(acc, fsem):
        pltpu.make_async_copy(acc, s_out.at[0], fsem).wait()

    def zero_acc(acc):
        z = jnp.zeros((16,), _f32)

        def zbody(j, _):
            for i in range(16):
                acc[pl.ds(j * 256 + i * 16, 16)] = z
            return 0

        lax.fori_loop(0, SROW // 256, zbody, 0)

    def geom(vl, g):
        eb = vl * DEG + g * 16
        ii = idx_v[pl.ds(eb, 16)]
        rx = plsc.load_gather(px_v, [ii]) - ox_v[pl.ds(vl, 16)][0]
        ry = plsc.load_gather(py_v, [ii]) - oy_v[pl.ds(vl, 16)][0]
        rz = plsc.load_gather(pz_v, [ii]) - oz_v[pl.ds(vl, 16)][0]
        r2 = rx * rx + ry * ry + rz * rz + 1e-12
        rnorm = _sqrt_newton(r2)
        linf = jnp.maximum(jnp.maximum(jnp.abs(rx), jnp.abs(ry)), jnp.abs(rz)) + 1e-12
        cc = rnorm / linf

        def axis_coords(r):
            q = jnp.minimum(jnp.maximum(r * cc, -1.0), 1.0)
            t = (q + 1.0) * (0.5 * (K - 1))
            t0 = jnp.minimum(t.astype(_i32), K - 2)
            f = jnp.minimum(jnp.maximum(t - t0.astype(_f32), 0.0), 1.0)
            return t0, f

        t0x, fx = axis_coords(rx)
        t0y, fy = axis_coords(ry)
        t0z, fz = axis_coords(rz)

        dd = di_v[pl.ds(eb, 16)]
        x6 = jnp.minimum(jnp.maximum(1.0 - dd * dd, 0.0), 1.0)
        imp = sc_v[pl.ds(eb, 16)] * (x6 * x6 * x6)

        wx = ((1.0 - fx) * imp, fx * imp)
        wy = (1.0 - fy, fy)
        wz = (1.0 - fz, fz)
        base = (t0x * (K * K) + t0y * K + t0z) * D
        lanes8 = lax.broadcasted_iota(_i32, (16,), 0) * 8 + (g * 128)
        for k, (dx, dy, dz) in enumerate(CORNERS):
            plsc.store_scatter(wbuf, [lanes8 + k], wx[dx] * wy[dy] * wz[dz])
            plsc.store_scatter(cbuf, [lanes8 + k],
                               base + (dx * (K * K * D) + dy * (K * D) + dz * D))
        return jnp.sum(imp)

    def accumulate(fbuf, acc):
        def pbody(p, _):
            wv = wbuf[pl.ds(p * 16, 16)]
            cv = cbuf[pl.ds(p * 16, 16)]
            for j in range(2):
                e = p * 2 + j
                fv = [fbuf[e, pl.ds(i * 16, 16)] for i in range(8)]
                for k in range(8):
                    w = wv[j * 8 + k]
                    co = cv[j * 8 + k]
                    for i in range(8):
                        plsc.addupdate(acc.at[pl.ds(co + i * 16, 16)], w * fv[i])
            return 0

        lax.fori_loop(0, DEG // 2, pbody, 0)

    NB = 4

    def run(vpt, v0):
        ept = vpt * DEG
        e0 = v0 * DEG
        pltpu.sync_copy(idxp.at[pl.ds(e0, ept)], idx_v.at[pl.ds(0, ept)])
        pltpu.sync_copy(scalep.at[pl.ds(e0, ept)], sc_v.at[pl.ds(0, ept)])
        pltpu.sync_copy(distp.at[pl.ds(e0, ept)], di_v.at[pl.ds(0, ept)])
        pltpu.sync_copy(ox.at[pl.ds(v0, vpt)], ox_v.at[pl.ds(0, vpt)])
        pltpu.sync_copy(oy.at[pl.ds(v0, vpt)], oy_v.at[pl.ds(0, vpt)])
        pltpu.sync_copy(oz.at[pl.ds(v0, vpt)], oz_v.at[pl.ds(0, vpt)])

        def process(it, j):
            vl = it * NB + j
            fbuf, acc = fbufs[j], accs[j]
            gsem, fsem = gsems[j], fsems[j]

            @pl.when(it > 0)
            def _():
                wait_flush(acc, fsem)

            zero_acc(acc)
            wait_gather(fbuf, gsem)
            nsum = geom(vl, 0) + geom(vl, 1)
            lane0 = lax.broadcasted_iota(_i32, (16,), 0) == 0
            plsc.store_scatter(nbuf, [jnp.zeros((16,), _i32) + vl],
                               jnp.zeros((16,), _f32) + nsum, mask=lane0)
            accumulate(fbuf, acc)
            pltpu.async_copy(acc, s_out.at[v0 + vl], fsem)

            @pl.when(vl + NB < vpt)
            def _():
                fire_gather(vl + NB, fbuf, gsem)

        for j in range(NB):
            fire_gather(j, fbufs[j], gsems[j])

        def vbody(it, _):
            for j in range(NB):
                process(it, j)
            return 0

        lax.fori_loop(0, vpt // NB, vbody, 0)

        for j in range(NB):
            wait_flush(accs[j], fsems[j])
        pltpu.sync_copy(nbuf.at[pl.ds(0, vpt)], norm_out.at[pl.ds(v0, vpt)])

    if vpt_a == vpt_b:
        run(vpt_a, (sid * NC + cid) * vpt_a)
    else:
        @pl.when(cid == 0)
        def _():
            run(vpt_a, sid * vpt_a)

        @pl.when(cid == 1)
        def _():
            run(vpt_b, NS * vpt_a + sid * vpt_b)


def _tc_body(s_ref, w_ref, n_ref, b_ref, o_ref):
    acc = jnp.dot(s_ref[...], w_ref[...], preferred_element_type=_f32)
    n = jnp.maximum(n_ref[...], 1e-8)
    o_ref[...] = jnp.maximum(acc / n + b_ref[...], 0.0)


def kernel(feats, inp_points, out_points, out_extents, scale_compat,
           neighbors_index, neighbors_distance, neighbors_row_splits, W, b):
    inv_r = 2.0 / out_extents[0]

    def col_pad(a, j, n):
        c = a[:, j] * inv_r
        return jnp.pad(c, (0, n - c.shape[0]))

    px = col_pad(inp_points, 0, VOXPAD)
    py = col_pad(inp_points, 1, VOXPAD)
    pz = col_pad(inp_points, 2, VOXPAD)
    ox = col_pad(out_points, 0, VOXPAD)
    oy = col_pad(out_points, 1, VOXPAD)
    oz = col_pad(out_points, 2, VOXPAD)

    idxp = jnp.pad(neighbors_index, (0, EPAD - E))
    scalep = jnp.pad(scale_compat, (0, EPAD - E))
    distp = jnp.pad(neighbors_distance, (0, EPAD - E))

    VPT_A = 360
    VPT_B = 640 - VPT_A
    VPT_MAX = max(VPT_A, VPT_B)
    sc_call = pl.kernel(
        functools.partial(_sc_body, VPT_A, VPT_B),
        mesh=plsc.VectorSubcoreMesh(core_axis_name="c", subcore_axis_name="s"),
        compiler_params=pltpu.CompilerParams(needs_layout_passes=False),
        out_type=[
            jax.ShapeDtypeStruct((VOXPAD, SROW), _f32),
            jax.ShapeDtypeStruct((VOXPAD,), _f32),
        ],
        scratch_types=[
            pltpu.VMEM((VPT_MAX * DEG,), _i32),
            pltpu.VMEM((VPT_MAX * DEG,), _f32),
            pltpu.VMEM((VPT_MAX * DEG,), _f32),
            pltpu.VMEM((VOXPAD,), _f32),
            pltpu.VMEM((VOXPAD,), _f32),
            pltpu.VMEM((VOXPAD,), _f32),
            pltpu.VMEM((VPT_MAX + 16,), _f32),
            pltpu.VMEM((VPT_MAX + 16,), _f32),
            pltpu.VMEM((VPT_MAX + 16,), _f32),
            [pltpu.VMEM((DEG, D), _f32) for _ in range(4)],
            [pltpu.VMEM((SROW,), _f32) for _ in range(4)],
            pltpu.VMEM((256,), _f32),
            pltpu.VMEM((256,), _i32),
            pltpu.VMEM((VPT_MAX,), _f32),
            [pltpu.SemaphoreType.DMA for _ in range(4)],
            [pltpu.SemaphoreType.DMA for _ in range(4)],
        ],
    )
    s, norm = sc_call(feats, idxp, scalep, distp, px, py, pz, ox, oy, oz)

    MBLK = 256
    out = pl.pallas_call(
        _tc_body,
        grid=(VOXPAD // MBLK,),
        in_specs=[
            pl.BlockSpec((MBLK, SROW), lambda i: (i, 0)),
            pl.BlockSpec((SROW, D), lambda i: (0, 0)),
            pl.BlockSpec((MBLK, 1), lambda i: (i, 0)),
            pl.BlockSpec((1, D), lambda i: (0, 0)),
        ],
        out_specs=pl.BlockSpec((MBLK, D), lambda i: (i, 0)),
        out_shape=jax.ShapeDtypeStruct((VOXPAD, D), _f32),
    )(s, W.reshape(SROW, D), norm.reshape(VOXPAD, 1), b.reshape(1, D))

    return out[:N_OUT]
